# baseline (speedup 1.0000x reference)
import dataclasses
import functools

import jax
import jax.numpy as jnp
from jax import lax
from jax.experimental import pallas as pl
from jax.experimental.pallas import tpu as pltpu
from jax.experimental.pallas import tpu_sc as plsc

NUM_USERS = 100000
NUM_ITEMS = 100000
EMB_DIM = 128
BATCH = 16384

NC, NS, L = 2, 16, 16
NW = NC * NS
B_PER_W = BATCH // NW
CHUNK = 64
NCHUNK = B_PER_W // CHUNK
NBUF = 3

_mesh = plsc.VectorSubcoreMesh(core_axis_name="c", subcore_axis_name="s")

_cp = pltpu.CompilerParams()
if "needs_layout_passes" in pltpu.CompilerParams.__dataclass_fields__:
    _cp = dataclasses.replace(_cp, needs_layout_passes=False)


@jax.jit
def _mf_forward(user_index, item_index, user_embedding, item_embedding):
    @functools.partial(
        pl.kernel,
        mesh=_mesh,
        compiler_params=_cp,
        out_type=jax.ShapeDtypeStruct((BATCH,), jnp.float32),
        scratch_types=[
            pltpu.VMEM((B_PER_W,), jnp.int32),
            pltpu.VMEM((B_PER_W,), jnp.int32),
            pltpu.VMEM((NBUF, CHUNK, EMB_DIM), jnp.float32),
            pltpu.VMEM((NBUF, CHUNK, EMB_DIM), jnp.float32),
            pltpu.VMEM((L, L + 1), jnp.float32),
            pltpu.VMEM((B_PER_W,), jnp.float32),
            pltpu.SemaphoreType.DMA((NBUF,)),
        ],
    )
    def k(uidx_hbm, iidx_hbm, utab_hbm, itab_hbm, out_hbm,
          uidx_v, iidx_v, u_v, i_v, acc_v, o_v, semr):
        wid = lax.axis_index("s") * NC + lax.axis_index("c")
        base = wid * B_PER_W
        pltpu.async_copy(uidx_hbm.at[pl.ds(base, B_PER_W)], uidx_v, semr.at[0])
        pltpu.async_copy(iidx_hbm.at[pl.ds(base, B_PER_W)], iidx_v, semr.at[1])
        pltpu.make_async_copy(
            uidx_hbm.at[pl.ds(base, B_PER_W)], uidx_v, semr.at[0]).wait()
        pltpu.make_async_copy(
            iidx_hbm.at[pl.ds(base, B_PER_W)], iidx_v, semr.at[1]).wait()

        def issue(g, sel):
            pltpu.async_copy(
                utab_hbm.at[uidx_v.at[pl.ds(g * CHUNK, CHUNK)]],
                u_v.at[sel], semr.at[sel])
            pltpu.async_copy(
                itab_hbm.at[iidx_v.at[pl.ds(g * CHUNK, CHUNK)]],
                i_v.at[sel], semr.at[sel])

        def drain(g, sel):
            pltpu.make_async_copy(
                utab_hbm.at[uidx_v.at[pl.ds(g * CHUNK, CHUNK)]],
                u_v.at[sel], semr.at[sel]).wait()
            pltpu.make_async_copy(
                itab_hbm.at[iidx_v.at[pl.ds(g * CHUNK, CHUNK)]],
                i_v.at[sel], semr.at[sel]).wait()

        rows = lax.iota(jnp.int32, L)

        for p in range(NBUF - 1):
            issue(p, p)

        @pl.loop(0, NCHUNK)
        def _(g):
            sel = lax.rem(g, NBUF)

            @pl.when(g + (NBUF - 1) < NCHUNK)
            def _():
                issue(g + (NBUF - 1), lax.rem(g + (NBUF - 1), NBUF))

            drain(g, sel)

            @pl.loop(0, CHUNK, step=L)
            def _(r0):
                @pl.loop(0, L, step=2)
                def _(j):
                    for jj in range(2):
                        r = r0 + j + jj
                        acc = (u_v[sel, r, pl.ds(0, L)]
                               * i_v[sel, r, pl.ds(0, L)])
                        for sg in range(1, EMB_DIM // L):
                            acc = acc + (u_v[sel, r, pl.ds(sg * L, L)]
                                         * i_v[sel, r, pl.ds(sg * L, L)])
                        acc_v[j + jj, pl.ds(0, L)] = acc
                cols = [plsc.load_gather(
                            acc_v, [rows, jnp.full((L,), c, jnp.int32)])
                        for c in range(L)]
                while len(cols) > 1:
                    cols = [cols[k] + cols[k + 1]
                            for k in range(0, len(cols) - 1, 2)] + (
                        [cols[-1]] if len(cols) % 2 else [])
                o_v[pl.ds(g * CHUNK + r0, L)] = cols[0]

        pltpu.sync_copy(o_v, out_hbm.at[pl.ds(base, B_PER_W)])

    return k(user_index, item_index, user_embedding, item_embedding)


def kernel(user_index, item_index, user_embedding, item_embedding):
    return _mf_forward(user_index.astype(jnp.int32),
                       item_index.astype(jnp.int32),
                       user_embedding, item_embedding)


# speedup vs baseline: 1.0245x; 1.0245x over previous
import dataclasses
import functools

import jax
import jax.numpy as jnp
from jax import lax
from jax.experimental import pallas as pl
from jax.experimental.pallas import tpu as pltpu
from jax.experimental.pallas import tpu_sc as plsc

NUM_USERS = 100000
NUM_ITEMS = 100000
EMB_DIM = 128
BATCH = 16384

NC, NS, L = 2, 16, 16
NW = NC * NS
B_PER_W = BATCH // NW
CHUNK = 64
NCHUNK = B_PER_W // CHUNK
NBUF = 2

_mesh = plsc.VectorSubcoreMesh(core_axis_name="c", subcore_axis_name="s")

_cp = pltpu.CompilerParams()
if "needs_layout_passes" in pltpu.CompilerParams.__dataclass_fields__:
    _cp = dataclasses.replace(_cp, needs_layout_passes=False)


@jax.jit
def _mf_forward(user_index, item_index, user_embedding, item_embedding):
    @functools.partial(
        pl.kernel,
        mesh=_mesh,
        compiler_params=_cp,
        out_type=jax.ShapeDtypeStruct((BATCH,), jnp.float32),
        scratch_types=[
            pltpu.VMEM((B_PER_W,), jnp.int32),
            pltpu.VMEM((B_PER_W,), jnp.int32),
            pltpu.VMEM((NBUF, CHUNK, EMB_DIM), jnp.float32),
            pltpu.VMEM((NBUF, CHUNK, EMB_DIM), jnp.float32),
            pltpu.VMEM((L, L + 1), jnp.float32),
            pltpu.VMEM((B_PER_W,), jnp.float32),
            pltpu.SemaphoreType.DMA((NBUF,)),
        ],
    )
    def k(uidx_hbm, iidx_hbm, utab_hbm, itab_hbm, out_hbm,
          uidx_v, iidx_v, u_v, i_v, acc_v, o_v, semr):
        wid = lax.axis_index("s") * NC + lax.axis_index("c")
        base = wid * B_PER_W
        pltpu.async_copy(uidx_hbm.at[pl.ds(base, B_PER_W)], uidx_v, semr.at[0])
        pltpu.async_copy(iidx_hbm.at[pl.ds(base, B_PER_W)], iidx_v, semr.at[1])
        pltpu.make_async_copy(
            uidx_hbm.at[pl.ds(base, B_PER_W)], uidx_v, semr.at[0]).wait()
        pltpu.make_async_copy(
            iidx_hbm.at[pl.ds(base, B_PER_W)], iidx_v, semr.at[1]).wait()

        def issue(g, sel):
            pltpu.async_copy(
                utab_hbm.at[uidx_v.at[pl.ds(g * CHUNK, CHUNK)]],
                u_v.at[sel], semr.at[sel])
            pltpu.async_copy(
                itab_hbm.at[iidx_v.at[pl.ds(g * CHUNK, CHUNK)]],
                i_v.at[sel], semr.at[sel])

        def drain(g, sel):
            pltpu.make_async_copy(
                utab_hbm.at[uidx_v.at[pl.ds(g * CHUNK, CHUNK)]],
                u_v.at[sel], semr.at[sel]).wait()
            pltpu.make_async_copy(
                itab_hbm.at[iidx_v.at[pl.ds(g * CHUNK, CHUNK)]],
                i_v.at[sel], semr.at[sel]).wait()

        rows = lax.iota(jnp.int32, L)

        for p in range(NBUF - 1):
            issue(p, p)

        @pl.loop(0, NCHUNK)
        def _(g):
            sel = lax.rem(g, NBUF)

            @pl.when(g + (NBUF - 1) < NCHUNK)
            def _():
                issue(g + (NBUF - 1), lax.rem(g + (NBUF - 1), NBUF))

            drain(g, sel)

            @pl.loop(0, CHUNK, step=L)
            def _(r0):
                @pl.loop(0, L, step=2)
                def _(j):
                    for jj in range(2):
                        r = r0 + j + jj
                        acc = (u_v[sel, r, pl.ds(0, L)]
                               * i_v[sel, r, pl.ds(0, L)])
                        for sg in range(1, EMB_DIM // L):
                            acc = acc + (u_v[sel, r, pl.ds(sg * L, L)]
                                         * i_v[sel, r, pl.ds(sg * L, L)])
                        acc_v[j + jj, pl.ds(0, L)] = acc
                cols = [plsc.load_gather(
                            acc_v, [rows, jnp.full((L,), c, jnp.int32)])
                        for c in range(L)]
                while len(cols) > 1:
                    cols = [cols[k] + cols[k + 1]
                            for k in range(0, len(cols) - 1, 2)] + (
                        [cols[-1]] if len(cols) % 2 else [])
                o_v[pl.ds(g * CHUNK + r0, L)] = cols[0]

        pltpu.sync_copy(o_v, out_hbm.at[pl.ds(base, B_PER_W)])

    return k(user_index, item_index, user_embedding, item_embedding)


def kernel(user_index, item_index, user_embedding, item_embedding):
    return _mf_forward(user_index.astype(jnp.int32),
                       item_index.astype(jnp.int32),
                       user_embedding, item_embedding)
